# baseline (speedup 1.0000x reference)
import functools

import jax
import jax.numpy as jnp
import numpy as np
from jax import lax
from jax.experimental import pallas as pl
from jax.experimental.pallas import tpu as pltpu
from jax.experimental.pallas import tpu_sc as plsc

VOXEL_SIZE = 0.04

_NW = 32
_CH = 128
_V = 9


@functools.partial(jax.jit, static_argnames=("n_rows", "n_cols"))
def _sc_gather_rows(table, idx, n_rows, n_cols):
    per_w = n_rows // _NW
    steps = per_w // _CH
    mesh = plsc.VectorSubcoreMesh(core_axis_name="c", subcore_axis_name="s")

    @functools.partial(
        pl.kernel,
        mesh=mesh,
        out_type=jax.ShapeDtypeStruct((n_rows, n_cols), jnp.float32),
        compiler_params=pltpu.CompilerParams(use_tc_tiling_on_sc=False),
        scratch_types=[
            pltpu.VMEM((_CH,), jnp.int32),
            pltpu.VMEM((_CH, n_cols), jnp.float32),
            pltpu.SemaphoreType.DMA,
        ],
    )
    def gather_kernel(table_hbm, idx_hbm, out_hbm, idx_v, rows_v, sem):
        cid = lax.axis_index("c")
        sid = lax.axis_index("s")
        wid = sid * 2 + cid
        base = wid * per_w

        def body(j, carry):
            off = base + j * _CH
            pltpu.sync_copy(idx_hbm.at[pl.ds(off, _CH)], idx_v)
            pltpu.async_copy(table_hbm.at[idx_v], rows_v, sem).wait()
            pltpu.sync_copy(rows_v, out_hbm.at[pl.ds(off, _CH)])
            return carry

        lax.fori_loop(0, steps, body, 0)

    return gather_kernel(table, idx)


def _back_project(coords, vol_origin, feats, KR):
    world = coords * VOXEL_SIZE + vol_origin[None, :]
    homog = jnp.concatenate([world, jnp.ones_like(world[:, :1])], axis=1)
    cam = jnp.einsum("vij,nj->vni", KR, homog)
    z = cam[..., 2]
    zs = jnp.maximum(z, 1e-6)
    px = cam[..., 0] / zs
    py = cam[..., 1] / zs
    V, C, H, W = feats.shape
    mask = (z > 0.1) & (px >= 0) & (px <= W - 1) & (py >= 0) & (py <= H - 1)
    ix = jnp.clip(jnp.round(px).astype(jnp.int32), 0, W - 1)
    iy = jnp.clip(jnp.round(py).astype(jnp.int32), 0, H - 1)
    lin = iy * W + ix

    Cp = ((C + 15) // 16) * 16
    table = feats.transpose(0, 2, 3, 1).reshape(V * H * W, C)
    if Cp != C:
        table = jnp.pad(table, ((0, 0), (0, Cp - C)))
    zero_row = V * H * W
    table = jnp.pad(table, ((0, 1), (0, 0)))

    N = coords.shape[0]
    Np = ((N + 4095) // 4096) * 4096
    offs = (jnp.arange(V, dtype=jnp.int32) * (H * W))[:, None]
    idx = jnp.where(mask, lin + offs, zero_row)
    if Np != N:
        idx = jnp.pad(idx, ((0, 0), (0, Np - N)), constant_values=zero_row)

    rows = _sc_gather_rows(table, idx.reshape(-1), V * Np, Cp)
    g = rows.reshape(V, Np, Cp)[:, :N, :C]
    count = mask.sum(axis=0).astype(jnp.float32)
    vol = g.sum(axis=0) / jnp.maximum(count[:, None], 1.0)
    return jnp.concatenate([vol, count[:, None]], axis=1)


def _generate_grid(n_vox, interval):
    r = jnp.arange(0, n_vox, interval, dtype=jnp.float32)
    gx, gy, gz = jnp.meshgrid(r, r, r, indexing="ij")
    return jnp.stack([gx.ravel(), gy.ravel(), gz.ravel()], axis=1)


def _upsample(prev_feat, prev_coords, interval):
    off = np.zeros((8, 3), dtype=np.float32)
    pos_list = [[0], [1], [2], [0, 1], [0, 2], [1, 2], [0, 1, 2]]
    for i, p in enumerate(pos_list):
        off[i + 1, p] = interval
    off = jnp.asarray(off)
    up_coords = (prev_coords[:, None, :] + off[None]).reshape(-1, 3)
    up_feat = jnp.repeat(prev_feat, 8, axis=0)
    return up_feat, up_coords


def kernel(feats_s0, feats_s1, feats_s2, proj_s0, proj_s1, proj_s2, vol_origin,
           W1_0, b1_0, W2_0, b2_0, Wt_0, bt_0, Wo_0, bo_0,
           W1_1, b1_1, W2_1, b2_1, Wt_1, bt_1, Wo_1, bo_1,
           W1_2, b1_2, W2_2, b2_2, Wt_2, bt_2, Wo_2, bo_2):
    p = dict(W1_0=W1_0, b1_0=b1_0, W2_0=W2_0, b2_0=b2_0,
             Wt_0=Wt_0, bt_0=bt_0, Wo_0=Wo_0, bo_0=bo_0,
             W1_1=W1_1, b1_1=b1_1, W2_1=W2_1, b2_1=b2_1,
             Wt_1=Wt_1, bt_1=bt_1, Wo_1=Wo_1, bo_1=bo_1,
             W1_2=W1_2, b1_2=b1_2, W2_2=W2_2, b2_2=b2_2,
             Wt_2=Wt_2, bt_2=bt_2, Wo_2=Wo_2, bo_2=bo_2)
    feats_all = [feats_s0, feats_s1, feats_s2]
    proj_all = [proj_s0, proj_s1, proj_s2]
    n_scales = 2
    prev_feat = None
    prev_coords = None
    out = None
    for i in range(3):
        interval = 2 ** (n_scales - i)
        scale = n_scales - i
        if i == 0:
            up_coords = _generate_grid(96, interval)
        else:
            up_feat, up_coords = _upsample(prev_feat, prev_coords, interval)
        volume = _back_project(up_coords, vol_origin, feats_all[scale], proj_all[scale])
        if i == 0:
            feat = volume
        else:
            feat = jnp.concatenate([volume, up_feat], axis=1)
        h = jax.nn.relu(feat @ p["W1_%d" % i] + p["b1_%d" % i])
        h = jax.nn.relu(h @ p["W2_%d" % i] + p["b2_%d" % i])
        tsdf = h @ p["Wt_%d" % i] + p["bt_%d" % i]
        occ = h @ p["Wo_%d" % i] + p["bo_%d" % i]
        kkeep = h.shape[0] // 2
        _, idx = jax.lax.top_k(occ[:, 0], kkeep)
        prev_coords = jnp.take(up_coords, idx, axis=0)
        prev_tsdf = jnp.take(tsdf, idx, axis=0)
        prev_occ = jnp.take(occ, idx, axis=0)
        keep_h = jnp.take(h, idx, axis=0)
        prev_feat = jnp.concatenate([keep_h, prev_tsdf, prev_occ], axis=1)
        if i == 2:
            out = (prev_coords, prev_tsdf)
    return out


# speedup vs baseline: 4.5951x; 4.5951x over previous
import functools

import jax
import jax.numpy as jnp
import numpy as np
from jax import lax
from jax.experimental import pallas as pl
from jax.experimental.pallas import tpu as pltpu
from jax.experimental.pallas import tpu_sc as plsc

VOXEL_SIZE = 0.04

_NW = 32
_CH = 128
_V = 9


@functools.partial(jax.jit, static_argnames=("n_rows", "n_cols"))
def _sc_gather_rows(table, idx, n_rows, n_cols):
    per_w = n_rows // _NW
    steps = per_w // _CH
    mesh = plsc.VectorSubcoreMesh(core_axis_name="c", subcore_axis_name="s")

    @functools.partial(
        pl.kernel,
        mesh=mesh,
        out_type=jax.ShapeDtypeStruct((n_rows, n_cols), jnp.float32),
        compiler_params=pltpu.CompilerParams(use_tc_tiling_on_sc=False),
        scratch_types=[
            pltpu.VMEM((_CH,), jnp.int32),
            pltpu.VMEM((_CH, n_cols), jnp.float32),
            pltpu.SemaphoreType.DMA,
        ],
    )
    def gather_kernel(table_hbm, idx_hbm, out_hbm, idx_v, rows_v, sem):
        cid = lax.axis_index("c")
        sid = lax.axis_index("s")
        wid = sid * 2 + cid
        base = wid * per_w

        def body(j, carry):
            off = base + j * _CH
            pltpu.sync_copy(idx_hbm.at[pl.ds(off, _CH)], idx_v)
            pltpu.async_copy(table_hbm.at[idx_v], rows_v, sem).wait()
            pltpu.sync_copy(rows_v, out_hbm.at[pl.ds(off, _CH)])
            return carry

        lax.fori_loop(0, steps, body, 0)

    return gather_kernel(table, idx)


def _back_project(coords, vol_origin, feats, KR):
    world = coords * VOXEL_SIZE + vol_origin[None, :]
    homog = jnp.concatenate([world, jnp.ones_like(world[:, :1])], axis=1)
    cam = jnp.einsum("vij,nj->vni", KR, homog)
    z = cam[..., 2]
    zs = jnp.maximum(z, 1e-6)
    px = cam[..., 0] / zs
    py = cam[..., 1] / zs
    V, C, H, W = feats.shape
    mask = (z > 0.1) & (px >= 0) & (px <= W - 1) & (py >= 0) & (py <= H - 1)
    ix = jnp.clip(jnp.round(px).astype(jnp.int32), 0, W - 1)
    iy = jnp.clip(jnp.round(py).astype(jnp.int32), 0, H - 1)
    lin = iy * W + ix

    Cp = ((C + 15) // 16) * 16
    table = feats.transpose(0, 2, 3, 1).reshape(V * H * W, C)
    if Cp != C:
        table = jnp.pad(table, ((0, 0), (0, Cp - C)))

    N = coords.shape[0]
    Np = ((N + 4095) // 4096) * 4096
    offs = (jnp.arange(V, dtype=jnp.int32) * (H * W))[:, None]
    idx = lin + offs
    if Np != N:
        idx = jnp.pad(idx, ((0, 0), (0, Np - N)))

    rows = _sc_gather_rows(table, idx.reshape(-1), V * Np, Cp)
    g = rows.reshape(V, Np, Cp)[:, :N, :C]
    g = g * mask[..., None].astype(g.dtype)
    count = mask.sum(axis=0).astype(jnp.float32)
    vol = g.sum(axis=0) / jnp.maximum(count[:, None], 1.0)
    return jnp.concatenate([vol, count[:, None]], axis=1)


def _generate_grid(n_vox, interval):
    r = jnp.arange(0, n_vox, interval, dtype=jnp.float32)
    gx, gy, gz = jnp.meshgrid(r, r, r, indexing="ij")
    return jnp.stack([gx.ravel(), gy.ravel(), gz.ravel()], axis=1)


def _upsample(prev_feat, prev_coords, interval):
    off = np.zeros((8, 3), dtype=np.float32)
    pos_list = [[0], [1], [2], [0, 1], [0, 2], [1, 2], [0, 1, 2]]
    for i, p in enumerate(pos_list):
        off[i + 1, p] = interval
    off = jnp.asarray(off)
    up_coords = (prev_coords[:, None, :] + off[None]).reshape(-1, 3)
    up_feat = jnp.repeat(prev_feat, 8, axis=0)
    return up_feat, up_coords


def kernel(feats_s0, feats_s1, feats_s2, proj_s0, proj_s1, proj_s2, vol_origin,
           W1_0, b1_0, W2_0, b2_0, Wt_0, bt_0, Wo_0, bo_0,
           W1_1, b1_1, W2_1, b2_1, Wt_1, bt_1, Wo_1, bo_1,
           W1_2, b1_2, W2_2, b2_2, Wt_2, bt_2, Wo_2, bo_2):
    p = dict(W1_0=W1_0, b1_0=b1_0, W2_0=W2_0, b2_0=b2_0,
             Wt_0=Wt_0, bt_0=bt_0, Wo_0=Wo_0, bo_0=bo_0,
             W1_1=W1_1, b1_1=b1_1, W2_1=W2_1, b2_1=b2_1,
             Wt_1=Wt_1, bt_1=bt_1, Wo_1=Wo_1, bo_1=bo_1,
             W1_2=W1_2, b1_2=b1_2, W2_2=W2_2, b2_2=b2_2,
             Wt_2=Wt_2, bt_2=bt_2, Wo_2=Wo_2, bo_2=bo_2)
    feats_all = [feats_s0, feats_s1, feats_s2]
    proj_all = [proj_s0, proj_s1, proj_s2]
    n_scales = 2
    prev_feat = None
    prev_coords = None
    out = None
    for i in range(3):
        interval = 2 ** (n_scales - i)
        scale = n_scales - i
        if i == 0:
            up_coords = _generate_grid(96, interval)
        else:
            up_feat, up_coords = _upsample(prev_feat, prev_coords, interval)
        volume = _back_project(up_coords, vol_origin, feats_all[scale], proj_all[scale])
        if i == 0:
            feat = volume
        else:
            feat = jnp.concatenate([volume, up_feat], axis=1)
        h = jax.nn.relu(feat @ p["W1_%d" % i] + p["b1_%d" % i])
        h = jax.nn.relu(h @ p["W2_%d" % i] + p["b2_%d" % i])
        tsdf = h @ p["Wt_%d" % i] + p["bt_%d" % i]
        occ = h @ p["Wo_%d" % i] + p["bo_%d" % i]
        kkeep = h.shape[0] // 2
        _, idx = jax.lax.top_k(occ[:, 0], kkeep)
        prev_coords = jnp.take(up_coords, idx, axis=0)
        prev_tsdf = jnp.take(tsdf, idx, axis=0)
        prev_occ = jnp.take(occ, idx, axis=0)
        keep_h = jnp.take(h, idx, axis=0)
        prev_feat = jnp.concatenate([keep_h, prev_tsdf, prev_occ], axis=1)
        if i == 2:
            out = (prev_coords, prev_tsdf)
    return out


# speedup vs baseline: 5.5702x; 1.2122x over previous
import functools

import jax
import jax.numpy as jnp
import numpy as np
from jax import lax
from jax.experimental import pallas as pl
from jax.experimental.pallas import tpu as pltpu
from jax.experimental.pallas import tpu_sc as plsc

VOXEL_SIZE = 0.04

_NW = 32
_CH = 256
_V = 9


@functools.partial(jax.jit, static_argnames=("n_rows", "n_cols"))
def _sc_gather_rows(table, idx, n_rows, n_cols):
    per_w = n_rows // _NW
    steps = per_w // _CH
    mesh = plsc.VectorSubcoreMesh(core_axis_name="c", subcore_axis_name="s")

    @functools.partial(
        pl.kernel,
        mesh=mesh,
        out_type=jax.ShapeDtypeStruct((n_rows, n_cols), jnp.float32),
        compiler_params=pltpu.CompilerParams(use_tc_tiling_on_sc=False),
        scratch_types=[
            pltpu.VMEM((_CH,), jnp.int32),
            pltpu.VMEM((_CH, n_cols), jnp.float32),
            pltpu.SemaphoreType.DMA,
        ],
    )
    def gather_kernel(table_hbm, idx_hbm, out_hbm, idx_v, rows_v, sem):
        cid = lax.axis_index("c")
        sid = lax.axis_index("s")
        wid = sid * 2 + cid
        base = wid * per_w

        def body(j, carry):
            off = base + j * _CH
            pltpu.sync_copy(idx_hbm.at[pl.ds(off, _CH)], idx_v)
            pltpu.async_copy(table_hbm.at[idx_v], rows_v, sem).wait()
            pltpu.sync_copy(rows_v, out_hbm.at[pl.ds(off, _CH)])
            return carry

        lax.fori_loop(0, steps, body, 0)

    return gather_kernel(table, idx)


def _back_project(coords, vol_origin, feats, KR):
    world = coords * VOXEL_SIZE + vol_origin[None, :]
    homog = jnp.concatenate([world, jnp.ones_like(world[:, :1])], axis=1)
    cam = jnp.einsum("vij,nj->vni", KR, homog)
    z = cam[..., 2]
    zs = jnp.maximum(z, 1e-6)
    px = cam[..., 0] / zs
    py = cam[..., 1] / zs
    V, C, H, W = feats.shape
    mask = (z > 0.1) & (px >= 0) & (px <= W - 1) & (py >= 0) & (py <= H - 1)
    ix = jnp.clip(jnp.round(px).astype(jnp.int32), 0, W - 1)
    iy = jnp.clip(jnp.round(py).astype(jnp.int32), 0, H - 1)
    lin = iy * W + ix

    Cp = ((C + 15) // 16) * 16
    table = feats.transpose(0, 2, 3, 1).reshape(V * H * W, C)
    if Cp != C:
        table = jnp.pad(table, ((0, 0), (0, Cp - C)))

    N = coords.shape[0]
    Np = ((N + 4095) // 4096) * 4096
    offs = (jnp.arange(V, dtype=jnp.int32) * (H * W))[:, None]
    idx = lin + offs
    if Np != N:
        spread = (jnp.arange(Np - N, dtype=jnp.int32) % (H * W))[None, :] + offs
        idx = jnp.concatenate([idx, spread + jnp.zeros((V, 1), jnp.int32)], axis=1)

    rows = _sc_gather_rows(table, idx.reshape(-1), V * Np, Cp)
    g = rows.reshape(V, Np, Cp)[:, :N, :C]
    g = g * mask[..., None].astype(g.dtype)
    count = mask.sum(axis=0).astype(jnp.float32)
    vol = g.sum(axis=0) / jnp.maximum(count[:, None], 1.0)
    return jnp.concatenate([vol, count[:, None]], axis=1)


def _generate_grid(n_vox, interval):
    r = jnp.arange(0, n_vox, interval, dtype=jnp.float32)
    gx, gy, gz = jnp.meshgrid(r, r, r, indexing="ij")
    return jnp.stack([gx.ravel(), gy.ravel(), gz.ravel()], axis=1)


def _upsample(prev_feat, prev_coords, interval):
    off = np.zeros((8, 3), dtype=np.float32)
    pos_list = [[0], [1], [2], [0, 1], [0, 2], [1, 2], [0, 1, 2]]
    for i, p in enumerate(pos_list):
        off[i + 1, p] = interval
    off = jnp.asarray(off)
    up_coords = (prev_coords[:, None, :] + off[None]).reshape(-1, 3)
    up_feat = jnp.repeat(prev_feat, 8, axis=0)
    return up_feat, up_coords


def kernel(feats_s0, feats_s1, feats_s2, proj_s0, proj_s1, proj_s2, vol_origin,
           W1_0, b1_0, W2_0, b2_0, Wt_0, bt_0, Wo_0, bo_0,
           W1_1, b1_1, W2_1, b2_1, Wt_1, bt_1, Wo_1, bo_1,
           W1_2, b1_2, W2_2, b2_2, Wt_2, bt_2, Wo_2, bo_2):
    p = dict(W1_0=W1_0, b1_0=b1_0, W2_0=W2_0, b2_0=b2_0,
             Wt_0=Wt_0, bt_0=bt_0, Wo_0=Wo_0, bo_0=bo_0,
             W1_1=W1_1, b1_1=b1_1, W2_1=W2_1, b2_1=b2_1,
             Wt_1=Wt_1, bt_1=bt_1, Wo_1=Wo_1, bo_1=bo_1,
             W1_2=W1_2, b1_2=b1_2, W2_2=W2_2, b2_2=b2_2,
             Wt_2=Wt_2, bt_2=bt_2, Wo_2=Wo_2, bo_2=bo_2)
    feats_all = [feats_s0, feats_s1, feats_s2]
    proj_all = [proj_s0, proj_s1, proj_s2]
    n_scales = 2
    prev_feat = None
    prev_coords = None
    out = None
    for i in range(3):
        interval = 2 ** (n_scales - i)
        scale = n_scales - i
        if i == 0:
            up_coords = _generate_grid(96, interval)
        else:
            up_feat, up_coords = _upsample(prev_feat, prev_coords, interval)
        volume = _back_project(up_coords, vol_origin, feats_all[scale], proj_all[scale])
        if i == 0:
            feat = volume
        else:
            feat = jnp.concatenate([volume, up_feat], axis=1)
        h = jax.nn.relu(feat @ p["W1_%d" % i] + p["b1_%d" % i])
        h = jax.nn.relu(h @ p["W2_%d" % i] + p["b2_%d" % i])
        tsdf = h @ p["Wt_%d" % i] + p["bt_%d" % i]
        occ = h @ p["Wo_%d" % i] + p["bo_%d" % i]
        kkeep = h.shape[0] // 2
        _, idx = jax.lax.top_k(occ[:, 0], kkeep)
        prev_coords = jnp.take(up_coords, idx, axis=0)
        prev_tsdf = jnp.take(tsdf, idx, axis=0)
        prev_occ = jnp.take(occ, idx, axis=0)
        keep_h = jnp.take(h, idx, axis=0)
        prev_feat = jnp.concatenate([keep_h, prev_tsdf, prev_occ], axis=1)
        if i == 2:
            out = (prev_coords, prev_tsdf)
    return out
